# Initial kernel scaffold; baseline (speedup 1.0000x reference)
import jax
import jax.numpy as jnp
from jax.experimental import pallas as pl


def kernel(x, ln1_g, ln1_b, ln2_g, ln2_b, Wq, bq, Wk, bk, Wv, bv, Wo, bo, Wr, br, sW1, sb1, sW2, sb2, rW1, rb1, rW2, rb2):
    raise NotImplementedError("write your pallas kernel here")


# speedup vs baseline: 1.1801x; 1.1801x over previous
import functools
import numpy as np
import jax
import jax.numpy as jnp
from jax.experimental import pallas as pl
from jax.experimental.pallas import tpu as pltpu

S = 2048
H = 768
NH, HD = 12, 64
NR = 14
NS = 2
TOPK = 2
INTER = 768
NRP = 128
RT = 256
AT = 512
SCALE = 1.0 / np.sqrt(HD)


def _gelu(x):
    return 0.5 * x * (1.0 + jax.lax.erf(x * np.float32(1.0 / np.sqrt(2.0))))


def _ln(x, g, b):
    m = jnp.mean(x, axis=-1, keepdims=True)
    v = jnp.mean((x - m) ** 2, axis=-1, keepdims=True)
    return (x - m) * jax.lax.rsqrt(v + 1e-5) * g + b


def _ln_qkv_body(x_ref, g_ref, b_ref, w_ref, bias_ref, o_ref):
    h = _ln(x_ref[...], g_ref[...], b_ref[...])
    o_ref[...] = jnp.dot(h, w_ref[...], preferred_element_type=jnp.float32) + bias_ref[...]


def _attn_body(q_ref, k_ref, v_ref, o_ref):
    q = q_ref[0]
    k = k_ref[0]
    s = jax.lax.dot_general(q, k, (((1,), (1,)), ((), ())),
                            preferred_element_type=jnp.float32) * SCALE
    m = jnp.max(s, axis=-1, keepdims=True)
    e = jnp.exp(s - m)
    p = e / jnp.sum(e, axis=-1, keepdims=True)
    o_ref[0] = jnp.dot(p, v_ref[0], preferred_element_type=jnp.float32)


def _proj_ln_body(c_ref, w_ref, b_ref, x_ref, g_ref, bb_ref, a_ref, h_ref):
    a = jnp.dot(c_ref[...], w_ref[...], preferred_element_type=jnp.float32)
    a = a + b_ref[...] + x_ref[...]
    a_ref[...] = a
    h_ref[...] = _ln(a, g_ref[...], bb_ref[...])


def _shared_router_body(h_ref, w1_ref, b1_ref, w2_ref, b2_ref, wr_ref, br_ref,
                        sh_ref, gates_ref):
    h = h_ref[...]
    act = _gelu(jnp.dot(h, w1_ref[...], preferred_element_type=jnp.float32) + b1_ref[...])
    sh_ref[...] = jnp.dot(act, w2_ref[...], preferred_element_type=jnp.float32) + b2_ref[...]
    logits = jnp.dot(h, wr_ref[...], preferred_element_type=jnp.float32) + br_ref[...]
    lm = jnp.max(logits, axis=-1, keepdims=True)
    ex = jnp.exp(logits - lm)
    aff = ex / jnp.sum(ex, axis=-1, keepdims=True)
    col = jax.lax.broadcasted_iota(jnp.int32, aff.shape, 1)
    i1 = jnp.argmax(aff, axis=-1)
    m1 = jnp.max(aff, axis=-1)
    masked = jnp.where(col == i1[:, None], -1.0, aff)
    i2 = jnp.argmax(masked, axis=-1)
    m2 = jnp.max(masked, axis=-1)
    gates_ref[...] = (m1[:, None] * (col == i1[:, None]).astype(jnp.float32)
                      + m2[:, None] * (col == i2[:, None]).astype(jnp.float32))


def _dense_moe_body(h_ref, g_ref, w1_ref, b1_ref, w2_ref, b2_ref,
                    a_ref, sh_ref, o_ref):
    e = pl.program_id(0)
    r = pl.program_id(1)
    x = h_ref[...]
    act = _gelu(jnp.dot(x, w1_ref[0], preferred_element_type=jnp.float32) + b1_ref[0])
    y = jnp.dot(act, w2_ref[0], preferred_element_type=jnp.float32) + b2_ref[0]
    gd = g_ref[...]
    col = jax.lax.broadcasted_iota(jnp.int32, gd.shape, 1)
    g = jnp.sum(jnp.where(col == e, gd, 0.0), axis=1, keepdims=True)
    contrib = y * g
    rows = pl.ds(r * RT, RT)

    @pl.when(e == 0)
    def _init():
        o_ref[rows, :] = contrib + a_ref[...] + sh_ref[...] + x

    @pl.when(e > 0)
    def _acc():
        o_ref[rows, :] = o_ref[rows, :] + contrib


def _ln_qkv(x2, g, b, wqkv, bqkv):
    return pl.pallas_call(
        _ln_qkv_body,
        grid=(S // RT,),
        in_specs=[
            pl.BlockSpec((RT, H), lambda i: (i, 0)),
            pl.BlockSpec((1, H), lambda i: (0, 0)),
            pl.BlockSpec((1, H), lambda i: (0, 0)),
            pl.BlockSpec((H, 3 * H), lambda i: (0, 0)),
            pl.BlockSpec((1, 3 * H), lambda i: (0, 0)),
        ],
        out_specs=pl.BlockSpec((RT, 3 * H), lambda i: (i, 0)),
        out_shape=jax.ShapeDtypeStruct((S, 3 * H), jnp.float32),
    )(x2, g, b, wqkv, bqkv)


def _attention(q, k, v):
    return pl.pallas_call(
        _attn_body,
        grid=(NH, S // AT),
        in_specs=[
            pl.BlockSpec((1, AT, HD), lambda h, r: (h, r, 0)),
            pl.BlockSpec((1, S, HD), lambda h, r: (h, 0, 0)),
            pl.BlockSpec((1, S, HD), lambda h, r: (h, 0, 0)),
        ],
        out_specs=pl.BlockSpec((1, AT, HD), lambda h, r: (h, r, 0)),
        out_shape=jax.ShapeDtypeStruct((NH, S, HD), jnp.float32),
    )(q, k, v)


def _proj_ln(ctx, wo, bo, x2, g2, b2):
    return pl.pallas_call(
        _proj_ln_body,
        grid=(S // RT,),
        in_specs=[
            pl.BlockSpec((RT, H), lambda i: (i, 0)),
            pl.BlockSpec((H, H), lambda i: (0, 0)),
            pl.BlockSpec((1, H), lambda i: (0, 0)),
            pl.BlockSpec((RT, H), lambda i: (i, 0)),
            pl.BlockSpec((1, H), lambda i: (0, 0)),
            pl.BlockSpec((1, H), lambda i: (0, 0)),
        ],
        out_specs=[
            pl.BlockSpec((RT, H), lambda i: (i, 0)),
            pl.BlockSpec((RT, H), lambda i: (i, 0)),
        ],
        out_shape=[
            jax.ShapeDtypeStruct((S, H), jnp.float32),
            jax.ShapeDtypeStruct((S, H), jnp.float32),
        ],
    )(ctx, wo, bo, x2, g2, b2)


def _shared_router(h2, w1c, b1c, w2c, b2s, wr, br):
    return pl.pallas_call(
        _shared_router_body,
        grid=(S // RT,),
        in_specs=[
            pl.BlockSpec((RT, H), lambda i: (i, 0)),
            pl.BlockSpec((H, NS * INTER), lambda i: (0, 0)),
            pl.BlockSpec((1, NS * INTER), lambda i: (0, 0)),
            pl.BlockSpec((NS * INTER, H), lambda i: (0, 0)),
            pl.BlockSpec((1, H), lambda i: (0, 0)),
            pl.BlockSpec((H, NRP), lambda i: (0, 0)),
            pl.BlockSpec((1, NRP), lambda i: (0, 0)),
        ],
        out_specs=[
            pl.BlockSpec((RT, H), lambda i: (i, 0)),
            pl.BlockSpec((RT, NRP), lambda i: (i, 0)),
        ],
        out_shape=[
            jax.ShapeDtypeStruct((S, H), jnp.float32),
            jax.ShapeDtypeStruct((S, NRP), jnp.float32),
        ],
    )(h2, w1c, b1c, w2c, b2s, wr, br)


def _dense_moe(h2, gates, rW1, rb1, rW2, rb2, a, shared):
    return pl.pallas_call(
        _dense_moe_body,
        grid=(NR, S // RT),
        in_specs=[
            pl.BlockSpec((RT, H), lambda e, r: (r, 0)),
            pl.BlockSpec((RT, NRP), lambda e, r: (r, 0)),
            pl.BlockSpec((1, H, INTER), lambda e, r: (e, 0, 0)),
            pl.BlockSpec((1, 1, INTER), lambda e, r: (e, 0, 0)),
            pl.BlockSpec((1, INTER, H), lambda e, r: (e, 0, 0)),
            pl.BlockSpec((1, 1, H), lambda e, r: (e, 0, 0)),
            pl.BlockSpec((RT, H), lambda e, r: (r, 0)),
            pl.BlockSpec((RT, H), lambda e, r: (r, 0)),
        ],
        out_specs=pl.BlockSpec((S, H), lambda e, r: (0, 0)),
        out_shape=jax.ShapeDtypeStruct((S, H), jnp.float32),
    )(h2, gates, rW1, rb1, rW2, rb2, a, shared)


def kernel(x, ln1_g, ln1_b, ln2_g, ln2_b, Wq, bq, Wk, bk, Wv, bv, Wo, bo,
           Wr, br, sW1, sb1, sW2, sb2, rW1, rb1, rW2, rb2):
    x2 = x[0]

    wqkv = jnp.concatenate([Wq, Wk, Wv], axis=1)
    bqkv = jnp.concatenate([bq, bk, bv])[None, :]
    qkv = _ln_qkv(x2, ln1_g[None, :], ln1_b[None, :], wqkv, bqkv)

    qkv3 = qkv.reshape(S, 3, NH, HD).transpose(1, 2, 0, 3)
    ctx = _attention(qkv3[0], qkv3[1], qkv3[2])
    ctx2 = ctx.transpose(1, 0, 2).reshape(S, NH * HD)

    a, h2 = _proj_ln(ctx2, Wo, bo[None, :], x2, ln2_g[None, :], ln2_b[None, :])

    w1c = jnp.concatenate([sW1[0], sW1[1]], axis=1)
    b1c = jnp.concatenate([sb1[0], sb1[1]])[None, :]
    w2c = jnp.concatenate([sW2[0], sW2[1]], axis=0)
    b2s = (sb2[0] + sb2[1])[None, :]
    wr_p = jnp.zeros((H, NRP), jnp.float32).at[:, :NR].set(Wr)
    br_p = jnp.full((NRP,), -1e30, jnp.float32).at[:NR].set(br)[None, :]
    shared, gates = _shared_router(h2, w1c, b1c, w2c, b2s, wr_p, br_p)

    out = _dense_moe(h2, gates, rW1, rb1[:, None, :], rW2, rb2[:, None, :],
                     a, shared)
    return out[None]
